# speedup vs baseline: 2.3556x; 2.3556x over previous
import functools

import jax
import jax.numpy as jnp
from jax.experimental import pallas as pl
from jax.experimental.pallas import tpu as pltpu

HIDDEN = 1024
FF = 4096
E = 8
TOPK = 2
NTOK = 2048

NEXP = NTOK * TOPK
B = 256
P = NEXP + E * B
NB = P // B
FFT = 2048
NF = FF // FFT


def _router_body(x_ref, rw_ref, gate_ref, idx_ref):
    logits = jnp.dot(x_ref[...], rw_ref[...], preferred_element_type=jnp.float32)
    m = jnp.max(logits, axis=-1, keepdims=True)
    p = jnp.exp(logits - m)
    probs = p / jnp.sum(p, axis=-1, keepdims=True)
    col = jax.lax.broadcasted_iota(jnp.int32, probs.shape, 1)
    g0 = jnp.max(probs, axis=-1)
    i0 = jnp.argmax(probs, axis=-1).astype(jnp.int32)
    probs2 = jnp.where(col == i0[:, None], -1.0, probs)
    g1 = jnp.max(probs2, axis=-1)
    i1 = jnp.argmax(probs2, axis=-1).astype(jnp.int32)
    gate_ref[...] = jnp.concatenate([g0[:, None], g1[:, None]], axis=-1)
    idx_ref[...] = jnp.concatenate([i0[:, None], i1[:, None]], axis=-1)


def _router(x, router_w):
    return pl.pallas_call(
        _router_body,
        out_shape=[
            jax.ShapeDtypeStruct((NTOK, TOPK), jnp.float32),
            jax.ShapeDtypeStruct((NTOK, TOPK), jnp.int32),
        ],
    )(x, router_w)


def _gemm_body(be_ref, px_ref, w1_ref, w2_ref, y_ref):
    f = pl.program_id(1)
    h = jax.nn.gelu(
        jnp.dot(px_ref[...], w1_ref[0], preferred_element_type=jnp.float32))
    part = jnp.dot(h, w2_ref[0], preferred_element_type=jnp.float32)

    @pl.when(f == 0)
    def _():
        y_ref[...] = part

    @pl.when(f != 0)
    def _():
        y_ref[...] += part


def _grouped_gemm(px, w1, w2, block_expert):
    grid_spec = pltpu.PrefetchScalarGridSpec(
        num_scalar_prefetch=1,
        grid=(NB, NF),
        in_specs=[
            pl.BlockSpec((B, HIDDEN), lambda b, f, be: (b, 0)),
            pl.BlockSpec((1, HIDDEN, FFT), lambda b, f, be: (be[b], 0, f)),
            pl.BlockSpec((1, FFT, HIDDEN), lambda b, f, be: (be[b], f, 0)),
        ],
        out_specs=pl.BlockSpec((B, HIDDEN), lambda b, f, be: (b, 0)),
    )
    return pl.pallas_call(
        _gemm_body,
        grid_spec=grid_spec,
        out_shape=jax.ShapeDtypeStruct((P, HIDDEN), jnp.float32),
    )(block_expert, px, w1, w2)


def kernel(x, router_w, w1, w2):
    gate, top_idx = _router(x, router_w)

    e = top_idx.reshape(-1)
    onehot = (e[:, None] == jnp.arange(E, dtype=jnp.int32)[None, :]).astype(jnp.int32)
    counts = jnp.sum(onehot, axis=0)
    pc = (counts + (B - 1)) // B * B
    padoff = jnp.cumsum(pc) - pc
    ranks = jnp.take_along_axis(jnp.cumsum(onehot, axis=0), e[:, None], axis=1)[:, 0] - 1
    dest = padoff[e] + ranks

    padend = padoff + pc
    bstart = jnp.arange(NB, dtype=jnp.int32) * B
    block_expert = jnp.minimum(
        jnp.sum((padend[None, :] <= bstart[:, None]).astype(jnp.int32), axis=1), E - 1
    ).astype(jnp.int32)

    row_token = jnp.zeros((P,), jnp.int32).at[dest].set(
        jnp.arange(NEXP, dtype=jnp.int32) // TOPK)
    px = x[row_token]

    y = _grouped_gemm(px, w1, w2, block_expert)

    y0 = y[dest[0::2]]
    y1 = y[dest[1::2]]
    return gate[:, 0:1] * y0 + gate[:, 1:2] * y1
